# baseline (speedup 1.0000x reference)
import functools

import jax
import jax.numpy as jnp
from jax import lax
from jax.experimental import pallas as pl
from jax.experimental.pallas import tpu as pltpu
from jax.experimental.pallas import tpu_sc as plsc

B = 16384
K = 32
NC = 2
NS = 16
NW = NC * NS
BPW = B // NW
CHUNK = 128
NCHUNK = BPW // CHUNK
GROUPS = CHUNK // 16


def _sc_kernel(users_hbm, jokes_hbm, U_hbm, V_hbm, a_hbm, b_hbm, g_hbm,
               out_hbm,
               idx_u, idx_j, u_rows, v_rows, a_v, b_v, g_v, out_v,
               *sems):
    wid = lax.axis_index("s") * NC + lax.axis_index("c")

    pltpu.sync_copy(users_hbm.at[pl.ds(wid * NCHUNK, NCHUNK)], idx_u)
    pltpu.sync_copy(jokes_hbm.at[pl.ds(wid * NCHUNK, NCHUNK)], idx_j)
    pltpu.sync_copy(g_hbm, g_v)

    handles = []
    for j in range(NCHUNK):
        h = []
        h.append(pltpu.async_copy(U_hbm.at[idx_u.at[j]], u_rows.at[j], sems[j]))
        h.append(pltpu.async_copy(V_hbm.at[idx_j.at[j]], v_rows.at[j], sems[j]))
        h.append(pltpu.async_copy(a_hbm.at[idx_u.at[j]], a_v.at[j], sems[j]))
        h.append(pltpu.async_copy(b_hbm.at[idx_j.at[j]], b_v.at[j], sems[j]))
        handles.append(h)

    lane = jnp.arange(16, dtype=jnp.int32)
    gvec = g_v[...]

    for j in range(NCHUNK):
        for h in handles[j]:
            h.wait()
        jfull = jnp.full((16,), j, dtype=jnp.int32)

        def group_body(grp, carry):
            row = lane + grp * 16
            acc = jnp.zeros((16,), dtype=jnp.float32)
            for k in range(K):
                kfull = jnp.full((16,), k, dtype=jnp.int32)
                uk = plsc.load_gather(u_rows, [jfull, row, kfull])
                vk = plsc.load_gather(v_rows, [jfull, row, kfull])
                acc = acc + uk * vk
            ab = a_v[j, pl.ds(grp * 16, 16)] + b_v[j, pl.ds(grp * 16, 16)]
            out_v[pl.ds(j * CHUNK + grp * 16, 16)] = acc + ab + gvec
            return carry

        lax.fori_loop(0, GROUPS, group_body, 0)

    pltpu.sync_copy(out_v, out_hbm.at[pl.ds(wid * BPW, BPW)])


@jax.jit
def _run(users2, jokes2, U, V, a_flat, b_flat, g16):
    mesh = plsc.VectorSubcoreMesh(core_axis_name="c", subcore_axis_name="s")
    f = functools.partial(
        pl.kernel,
        mesh=mesh,
        out_type=jax.ShapeDtypeStruct((B,), jnp.float32),
        scratch_types=[
            pltpu.VMEM((NCHUNK, CHUNK), jnp.int32),
            pltpu.VMEM((NCHUNK, CHUNK), jnp.int32),
            pltpu.VMEM((NCHUNK, CHUNK, K), jnp.float32),
            pltpu.VMEM((NCHUNK, CHUNK, K), jnp.float32),
            pltpu.VMEM((NCHUNK, CHUNK), jnp.float32),
            pltpu.VMEM((NCHUNK, CHUNK), jnp.float32),
            pltpu.VMEM((16,), jnp.float32),
            pltpu.VMEM((BPW,), jnp.float32),
        ] + [pltpu.SemaphoreType.DMA] * NCHUNK,
        compiler_params=pltpu.CompilerParams(
            needs_layout_passes=False, use_tc_tiling_on_sc=False
        ),
    )(_sc_kernel)
    return f(users2, jokes2, U, V, a_flat, b_flat, g16)


def kernel(users, jokes, U, V, a, b, g):
    users2 = users.astype(jnp.int32).reshape(B // CHUNK, CHUNK)
    jokes2 = jokes.astype(jnp.int32).reshape(B // CHUNK, CHUNK)
    a_flat = a.reshape(-1)
    b_flat = b.reshape(-1)
    g16 = jnp.broadcast_to(g.astype(jnp.float32), (16,))
    return _run(users2, jokes2, U, V, a_flat, b_flat, g16)
